# baseline (speedup 1.0000x reference)
import jax
import jax.numpy as jnp
from jax import lax
from jax.experimental import pallas as pl
from jax.experimental.pallas import tpu as pltpu
from jax.experimental.pallas import tpu_sc as plsc

USER_NUM = 100000
ITEM_NUM = 50000
TAG_NUM = 20
EMB = 16
BATCH = 4096

NC = 2
NS = 16
L = 16
NW = NC * NS
BPW = BATCH // NW
NGRP = BATCH // L
F = TAG_NUM * EMB
ROW = F + 4 * TAG_NUM
SST = 416

NFULL_U = USER_NUM // 128
NFULL_I = ITEM_NUM // 128
TAILB_U = NFULL_U * 128
TAILB_I = NFULL_I * 128
NCOL_U = NFULL_U + 1
NCOL_I = NFULL_I + 1


def _fire_slabs(embT, b0T, b1T, b2T, b3T, m, tile_v, ball_v, semt):
    off = pl.multiple_of(m * 128, 128)
    cps = [
        pltpu.make_async_copy(embT.at[:, pl.ds(off, 128)], tile_v, semt),
        pltpu.make_async_copy(b0T.at[:, pl.ds(off, 128)],
                              ball_v.at[pl.ds(0, TAG_NUM), :], semt),
        pltpu.make_async_copy(b1T.at[:, pl.ds(off, 128)],
                              ball_v.at[pl.ds(24, TAG_NUM), :], semt),
        pltpu.make_async_copy(b2T.at[:, pl.ds(off, 128)],
                              ball_v.at[pl.ds(48, TAG_NUM), :], semt),
        pltpu.make_async_copy(b3T.at[:, pl.ds(off, 128)],
                              ball_v.at[pl.ds(72, TAG_NUM), :], semt),
    ]
    for cp in cps:
        cp.start()
    return cps


def _wait_slabs(embT, b0T, b1T, b2T, b3T, tile_v, ball_v, semt):
    cps = [
        pltpu.make_async_copy(embT.at[:, pl.ds(0, 128)], tile_v, semt),
        pltpu.make_async_copy(b0T.at[:, pl.ds(0, 128)],
                              ball_v.at[pl.ds(0, TAG_NUM), :], semt),
        pltpu.make_async_copy(b1T.at[:, pl.ds(0, 128)],
                              ball_v.at[pl.ds(24, TAG_NUM), :], semt),
        pltpu.make_async_copy(b2T.at[:, pl.ds(0, 128)],
                              ball_v.at[pl.ds(48, TAG_NUM), :], semt),
        pltpu.make_async_copy(b3T.at[:, pl.ds(0, 128)],
                              ball_v.at[pl.ds(72, TAG_NUM), :], semt),
    ]
    for cp in cps:
        cp.wait()


def _gather_side(idx_v, embT, b0T, b1T, b2T, b3T, tail_hbm, out_hbm,
                 cand_b, cand_u, tile0, ball0, tile1, ball1,
                 stage_v, semt0, semt1, semA, semB,
                 wid, ncol, nfull, tailbase):
    lanes = lax.iota(jnp.int32, L)
    lo = (wid * ncol) // NW
    hi = ((wid + 1) * ncol) // NW

    def cgroup(g, cnt_vec):
        uv = idx_v[pl.ds(g * L, L)]
        tc = lax.shift_right_logical(uv, 7)
        mask = (tc >= lo) & (tc < hi)
        inc = plsc.cumsum(mask.astype(jnp.int32))
        pos = cnt_vec + inc - 1
        plsc.store_scatter(cand_b, [pos], lanes + g * L, mask=mask)
        plsc.store_scatter(cand_u, [pos], uv, mask=mask)
        return cnt_vec + plsc.all_reduce_population_count(mask)

    cnt_vec = lax.fori_loop(0, NGRP, cgroup, jnp.zeros((L,), jnp.int32))
    cnt = jnp.max(cnt_vec)
    cnt_splat = jnp.full((L,), cnt, jnp.int32)
    nchunks = lax.shift_right_logical(cnt + 15, 4)

    def chunk_scan(m, is_tail, tile_v, ball_v):
        def chunk_body(c, pend2):
            bv = cand_b[pl.ds(c * L, L)]
            uvv = cand_u[pl.ds(c * L, L)]
            mask0 = (lax.shift_right_logical(uvv, 7) == m)
            mask0 = mask0 & ((lanes + c * L) < cnt_splat)
            nm = jnp.max(plsc.all_reduce_population_count(mask0))

            def match_body(i, st):
                mask, p0, p1 = st
                ffs = plsc.all_reduce_ffs(mask)
                sel = lanes == ffs
                zero = jnp.zeros((L,), jnp.int32)
                b_s = jnp.max(jnp.where(sel, bv, zero))
                u_s = jnp.max(jnp.where(sel, uvv, zero))
                slot = i & 1
                sbase = slot * SST

                @pl.when((slot == 0) & (p0 == 1))
                def _():
                    pltpu.make_async_copy(
                        stage_v.at[pl.ds(0, ROW)],
                        out_hbm.at[pl.ds(0, ROW)], semA).wait()

                @pl.when((slot == 1) & (p1 == 1))
                def _():
                    pltpu.make_async_copy(
                        stage_v.at[pl.ds(SST, ROW)],
                        out_hbm.at[pl.ds(0, ROW)], semB).wait()

                uin = jnp.full((L,), u_s & 127, jnp.int32)

                if not is_tail:
                    for c2 in range(F // L):
                        fv = lanes + c2 * L
                        stage_v[pl.ds(sbase + c2 * L, L)] = (
                            plsc.load_gather(tile_v, [fv, uin]))
                    for s in range(5):
                        w = lanes + 16 * s
                        k = ((w >= 20).astype(jnp.int32)
                             + (w >= 40).astype(jnp.int32)
                             + (w >= 60).astype(jnp.int32))
                        stage_v[pl.ds(sbase + F + 16 * s, L)] = (
                            plsc.load_gather(ball_v, [w + 4 * k, uin]))
                    @pl.when(slot == 0)
                    def _():
                        pltpu.make_async_copy(
                            stage_v.at[pl.ds(0, ROW)],
                            out_hbm.at[pl.ds(b_s * ROW, ROW)], semA).start()

                    @pl.when(slot == 1)
                    def _():
                        pltpu.make_async_copy(
                            stage_v.at[pl.ds(SST, ROW)],
                            out_hbm.at[pl.ds(b_s * ROW, ROW)], semB).start()

                    np0 = jnp.where(slot == 0, 1, p0)
                    np1 = jnp.where(slot == 1, 1, p1)
                else:
                    r = u_s - tailbase
                    pltpu.sync_copy(tail_hbm.at[pl.ds(r * ROW, ROW)],
                                    stage_v.at[pl.ds(0, ROW)])
                    pltpu.sync_copy(stage_v.at[pl.ds(0, ROW)],
                                    out_hbm.at[pl.ds(b_s * ROW, ROW)])
                    np0, np1 = p0, p1

                return (mask & jnp.logical_not(sel), np0, np1)

            mask_f, p0f, p1f = lax.fori_loop(
                0, nm, match_body, (mask0, jnp.int32(0), jnp.int32(0)))

            @pl.when(p0f == 1)
            def _():
                pltpu.make_async_copy(stage_v.at[pl.ds(0, ROW)],
                                      out_hbm.at[pl.ds(0, ROW)], semA).wait()

            @pl.when(p1f == 1)
            def _():
                pltpu.make_async_copy(stage_v.at[pl.ds(SST, ROW)],
                                      out_hbm.at[pl.ds(0, ROW)], semB).wait()

            return pend2

        lax.fori_loop(0, nchunks, chunk_body, 0)

    _fire_slabs(embT, b0T, b1T, b2T, b3T, lo, tile0, ball0, semt0)

    def col_body(m, carry):
        is_tail_col = m >= nfull
        par = (m - lo) & 1
        nxt = m + 1
        do_pre = (nxt < hi) & (nxt < nfull)

        @pl.when(jnp.logical_not(is_tail_col) & (par == 0))
        def _():
            _wait_slabs(embT, b0T, b1T, b2T, b3T, tile0, ball0, semt0)

        @pl.when(jnp.logical_not(is_tail_col) & (par == 1))
        def _():
            _wait_slabs(embT, b0T, b1T, b2T, b3T, tile1, ball1, semt1)

        @pl.when(do_pre & (par == 0))
        def _():
            _fire_slabs(embT, b0T, b1T, b2T, b3T, nxt, tile1, ball1, semt1)

        @pl.when(do_pre & (par == 1))
        def _():
            _fire_slabs(embT, b0T, b1T, b2T, b3T, nxt, tile0, ball0, semt0)

        @pl.when(is_tail_col)
        def _():
            chunk_scan(m, True, tile0, ball0)

        @pl.when(jnp.logical_not(is_tail_col) & (par == 0))
        def _():
            chunk_scan(m, False, tile0, ball0)

        @pl.when(jnp.logical_not(is_tail_col) & (par == 1))
        def _():
            chunk_scan(m, False, tile1, ball1)

        return carry

    lax.fori_loop(lo, hi, col_body, 0)


def _phase1_body(user_hbm, item_hbm, uembT, iembT,
                 urbT, uvbT, uibT, ucfT, irbT, ivbT, iibT, icfT,
                 utail_hbm, itail_hbm, uout_hbm, iout_hbm,
                 uidx_v, iidx_v, cand_b, cand_u,
                 tile0, ball0, tile1, ball1, stage_v,
                 semt0, semt1, semA, semB):
    wid = lax.axis_index("s") * NC + lax.axis_index("c")

    pltpu.sync_copy(user_hbm, uidx_v)
    pltpu.sync_copy(item_hbm, iidx_v)

    _gather_side(uidx_v, uembT, urbT, uvbT, uibT, ucfT, utail_hbm, uout_hbm,
                 cand_b, cand_u, tile0, ball0, tile1, ball1,
                 stage_v, semt0, semt1, semA, semB,
                 wid, NCOL_U, NFULL_U, TAILB_U)
    _gather_side(iidx_v, iembT, irbT, ivbT, iibT, icfT, itail_hbm, iout_hbm,
                 cand_b, cand_u, tile0, ball0, tile1, ball1,
                 stage_v, semt0, semt1, semA, semB,
                 wid, NCOL_I, NFULL_I, TAILB_I)


def _phase2_body(uout_hbm, iout_hbm, g_hbm, out_hbm,
                 urows_v, irows_v, g_v, out_v, sem, sem2):
    wid = lax.axis_index("s") * NC + lax.axis_index("c")
    base = wid * BPW
    HALF = BPW * ROW // 2

    cps1 = [
        pltpu.make_async_copy(uout_hbm.at[pl.ds(base * ROW, HALF)],
                              urows_v.at[pl.ds(0, HALF)], sem),
        pltpu.make_async_copy(iout_hbm.at[pl.ds(base * ROW, HALF)],
                              irows_v.at[pl.ds(0, HALF)], sem),
        pltpu.make_async_copy(g_hbm, g_v, sem),
    ]
    for cp in cps1:
        cp.start()
    cps2 = [
        pltpu.make_async_copy(uout_hbm.at[pl.ds(base * ROW + HALF, HALF)],
                              urows_v.at[pl.ds(HALF, HALF)], sem2),
        pltpu.make_async_copy(iout_hbm.at[pl.ds(base * ROW + HALF, HALF)],
                              irows_v.at[pl.ds(HALF, HALF)], sem2),
    ]
    for cp in cps2:
        cp.start()
    for cp in cps1:
        cp.wait()

    lanes = lax.iota(jnp.int32, L)

    def group_body(g, carry):
        jbase = (lanes + g * L) * ROW

        def tag_body(t, racc):
            tcol = jnp.full((L,), t, dtype=jnp.int32)
            dot = jnp.zeros((L,), jnp.float32)
            for e in range(EMB):
                col = jbase + tcol * EMB + e
                uv = plsc.load_gather(urows_v, [col])
                iv = plsc.load_gather(irows_v, [col])
                dot = dot + uv * iv

            def gscalar(k):
                krow = jnp.full((L,), k, dtype=jnp.int32)
                return plsc.load_gather(g_v, [krow * TAG_NUM + tcol])

            score = jnp.zeros((L,), jnp.float32)
            for k in range(3):
                bcol = jbase + F + TAG_NUM * k + tcol
                x = (dot + plsc.load_gather(urows_v, [bcol])
                     + plsc.load_gather(irows_v, [bcol]) + gscalar(k))
                score = score + 1.0 / (1.0 + jnp.exp(-x))
            score = score * (1.0 / 3.0)

            ccol = jbase + F + TAG_NUM * 3 + tcol
            coeff = (plsc.load_gather(urows_v, [ccol])
                     + plsc.load_gather(irows_v, [ccol]) + gscalar(3))
            return racc + score * coeff

        racc = lax.fori_loop(0, TAG_NUM, tag_body, jnp.zeros((L,), jnp.float32))
        out_v[pl.ds(g * L, L)] = racc
        return carry

    lax.fori_loop(0, BPW // L // 2, group_body, 0)
    for cp in cps2:
        cp.wait()
    lax.fori_loop(BPW // L // 2, BPW // L, group_body, 0)
    pltpu.sync_copy(out_v, out_hbm.at[pl.ds(base, BPW)])


def _make_phase1():
    mesh = plsc.VectorSubcoreMesh(core_axis_name="c", subcore_axis_name="s")
    return pl.kernel(
        _phase1_body,
        out_type=(jax.ShapeDtypeStruct((BATCH * ROW,), jnp.float32),
                  jax.ShapeDtypeStruct((BATCH * ROW,), jnp.float32)),
        mesh=mesh,
        scratch_types=[
            pltpu.VMEM((BATCH,), jnp.int32),
            pltpu.VMEM((BATCH,), jnp.int32),
            pltpu.VMEM((BATCH,), jnp.int32),
            pltpu.VMEM((BATCH,), jnp.int32),
            pltpu.VMEM((F, 128), jnp.float32),
            pltpu.VMEM((96, 128), jnp.float32),
            pltpu.VMEM((F, 128), jnp.float32),
            pltpu.VMEM((96, 128), jnp.float32),
            pltpu.VMEM((2 * SST,), jnp.float32),
            pltpu.SemaphoreType.DMA,
            pltpu.SemaphoreType.DMA,
            pltpu.SemaphoreType.DMA,
            pltpu.SemaphoreType.DMA,
        ],
        compiler_params=pltpu.CompilerParams(
            use_tc_tiling_on_sc=True, needs_layout_passes=False),
    )


def _make_phase2():
    mesh = plsc.VectorSubcoreMesh(core_axis_name="c", subcore_axis_name="s")
    return pl.kernel(
        _phase2_body,
        out_type=jax.ShapeDtypeStruct((BATCH,), jnp.float32),
        mesh=mesh,
        scratch_types=[
            pltpu.VMEM((BPW * ROW,), jnp.float32),
            pltpu.VMEM((BPW * ROW,), jnp.float32),
            pltpu.VMEM((4 * TAG_NUM,), jnp.float32),
            pltpu.VMEM((BPW,), jnp.float32),
            pltpu.SemaphoreType.DMA,
            pltpu.SemaphoreType.DMA,
        ],
        compiler_params=pltpu.CompilerParams(
            use_tc_tiling_on_sc=False, needs_layout_passes=False),
    )


@jax.jit
def _sulm(user, item, uemb3, iemb3, urb, irb, uvb, ivb, uib, iib, ucf, icf, g4):
    uembT = jnp.transpose(uemb3, (1, 2, 0)).reshape(F, USER_NUM)
    iembT = jnp.transpose(iemb3, (1, 2, 0)).reshape(F, ITEM_NUM)

    def tails(emb3, b0, b1, b2, b3, tb, n):
        return jnp.concatenate(
            [emb3[tb:].reshape(n - tb, F), b0[tb:], b1[tb:], b2[tb:], b3[tb:]],
            axis=1).reshape(-1)

    utail = tails(uemb3, urb, uvb, uib, ucf, TAILB_U, USER_NUM)
    itail = tails(iemb3, irb, ivb, iib, icf, TAILB_I, ITEM_NUM)

    uout, iout = _make_phase1()(
        user, item, uembT, iembT,
        urb.T, uvb.T, uib.T, ucf.T, irb.T, ivb.T, iib.T, icf.T,
        utail, itail)
    return _make_phase2()(uout, iout, g4.reshape(-1))


def kernel(user, item,
           user_tag_embeddings, item_tag_embeddings,
           user_reason_bias, item_reason_bias, global_reason_bias,
           user_video_bias, item_video_bias, global_video_bias,
           user_interest_bias, item_interest_bias, global_interest_bias,
           user_coeff, item_coeff, global_coeff):
    g4 = jnp.concatenate(
        [global_reason_bias, global_video_bias, global_interest_bias,
         global_coeff], axis=0)
    return _sulm(user, item, user_tag_embeddings, item_tag_embeddings,
                 user_reason_bias, item_reason_bias,
                 user_video_bias, item_video_bias,
                 user_interest_bias, item_interest_bias,
                 user_coeff, item_coeff, g4)


# speedup vs baseline: 1.0058x; 1.0058x over previous
import jax
import jax.numpy as jnp
from jax import lax
from jax.experimental import pallas as pl
from jax.experimental.pallas import tpu as pltpu
from jax.experimental.pallas import tpu_sc as plsc

USER_NUM = 100000
ITEM_NUM = 50000
TAG_NUM = 20
EMB = 16
BATCH = 4096

NC = 2
NS = 16
L = 16
NW = NC * NS
BPW = BATCH // NW
NGRP = BATCH // L
F = TAG_NUM * EMB
ROW = F + 4 * TAG_NUM
SST = 416

NFULL_U = USER_NUM // 128
NFULL_I = ITEM_NUM // 128
TAILB_U = NFULL_U * 128
TAILB_I = NFULL_I * 128
NCOL_U = NFULL_U + 1
NCOL_I = NFULL_I + 1


def _fire_slabs(embT, b0T, b1T, b2T, b3T, m, tile_v, ball_v, semt):
    off = pl.multiple_of(m * 128, 128)
    cps = [
        pltpu.make_async_copy(embT.at[:, pl.ds(off, 128)], tile_v, semt),
        pltpu.make_async_copy(b0T.at[:, pl.ds(off, 128)],
                              ball_v.at[pl.ds(0, TAG_NUM), :], semt),
        pltpu.make_async_copy(b1T.at[:, pl.ds(off, 128)],
                              ball_v.at[pl.ds(24, TAG_NUM), :], semt),
        pltpu.make_async_copy(b2T.at[:, pl.ds(off, 128)],
                              ball_v.at[pl.ds(48, TAG_NUM), :], semt),
        pltpu.make_async_copy(b3T.at[:, pl.ds(off, 128)],
                              ball_v.at[pl.ds(72, TAG_NUM), :], semt),
    ]
    for cp in cps:
        cp.start()
    return cps


def _wait_slabs(embT, b0T, b1T, b2T, b3T, tile_v, ball_v, semt):
    cps = [
        pltpu.make_async_copy(embT.at[:, pl.ds(0, 128)], tile_v, semt),
        pltpu.make_async_copy(b0T.at[:, pl.ds(0, 128)],
                              ball_v.at[pl.ds(0, TAG_NUM), :], semt),
        pltpu.make_async_copy(b1T.at[:, pl.ds(0, 128)],
                              ball_v.at[pl.ds(24, TAG_NUM), :], semt),
        pltpu.make_async_copy(b2T.at[:, pl.ds(0, 128)],
                              ball_v.at[pl.ds(48, TAG_NUM), :], semt),
        pltpu.make_async_copy(b3T.at[:, pl.ds(0, 128)],
                              ball_v.at[pl.ds(72, TAG_NUM), :], semt),
    ]
    for cp in cps:
        cp.wait()


def _gather_side(idx_v, embT, b0T, b1T, b2T, b3T, tail_hbm, out_hbm,
                 cand_b, cand_u, tile0, ball0, tile1, ball1,
                 stage_v, semt0, semt1, semA, semB,
                 wid, ncol, nfull, tailbase):
    lanes = lax.iota(jnp.int32, L)
    lo = (wid * ncol) // NW
    hi = ((wid + 1) * ncol) // NW

    def cgroup(g, cnt_vec):
        uv = idx_v[pl.ds(g * L, L)]
        tc = lax.shift_right_logical(uv, 7)
        mask = (tc >= lo) & (tc < hi)
        inc = plsc.cumsum(mask.astype(jnp.int32))
        pos = cnt_vec + inc - 1
        plsc.store_scatter(cand_b, [pos], lanes + g * L, mask=mask)
        plsc.store_scatter(cand_u, [pos], uv, mask=mask)
        return cnt_vec + plsc.all_reduce_population_count(mask)

    cnt_vec = lax.fori_loop(0, NGRP, cgroup, jnp.zeros((L,), jnp.int32))
    cnt = jnp.max(cnt_vec)
    cnt_splat = jnp.full((L,), cnt, jnp.int32)
    nchunks = lax.shift_right_logical(cnt + 15, 4)

    def chunk_scan(m, is_tail, tile_v, ball_v):
        def chunk_body(c, pend2):
            bv = cand_b[pl.ds(c * L, L)]
            uvv = cand_u[pl.ds(c * L, L)]
            mask0 = (lax.shift_right_logical(uvv, 7) == m)
            mask0 = mask0 & ((lanes + c * L) < cnt_splat)
            nm = jnp.max(plsc.all_reduce_population_count(mask0))

            def match_body(i, st):
                mask, p0, p1 = st
                ffs = plsc.all_reduce_ffs(mask)
                sel = lanes == ffs
                zero = jnp.zeros((L,), jnp.int32)
                b_s = jnp.max(jnp.where(sel, bv, zero))
                u_s = jnp.max(jnp.where(sel, uvv, zero))
                slot = i & 1
                sbase = slot * SST

                @pl.when((slot == 0) & (p0 == 1))
                def _():
                    pltpu.make_async_copy(
                        stage_v.at[pl.ds(0, ROW)],
                        out_hbm.at[pl.ds(0, ROW)], semA).wait()

                @pl.when((slot == 1) & (p1 == 1))
                def _():
                    pltpu.make_async_copy(
                        stage_v.at[pl.ds(SST, ROW)],
                        out_hbm.at[pl.ds(0, ROW)], semB).wait()

                uin = jnp.full((L,), u_s & 127, jnp.int32)

                if not is_tail:
                    for c2 in range(F // L):
                        fv = lanes + c2 * L
                        stage_v[pl.ds(sbase + c2 * L, L)] = (
                            plsc.load_gather(tile_v, [fv, uin]))
                    for s in range(5):
                        w = lanes + 16 * s
                        k = ((w >= 20).astype(jnp.int32)
                             + (w >= 40).astype(jnp.int32)
                             + (w >= 60).astype(jnp.int32))
                        stage_v[pl.ds(sbase + F + 16 * s, L)] = (
                            plsc.load_gather(ball_v, [w + 4 * k, uin]))
                    @pl.when(slot == 0)
                    def _():
                        pltpu.make_async_copy(
                            stage_v.at[pl.ds(0, ROW)],
                            out_hbm.at[pl.ds(b_s * ROW, ROW)], semA).start()

                    @pl.when(slot == 1)
                    def _():
                        pltpu.make_async_copy(
                            stage_v.at[pl.ds(SST, ROW)],
                            out_hbm.at[pl.ds(b_s * ROW, ROW)], semB).start()

                    np0 = jnp.where(slot == 0, 1, p0)
                    np1 = jnp.where(slot == 1, 1, p1)
                else:
                    r = u_s - tailbase
                    pltpu.sync_copy(tail_hbm.at[pl.ds(r * ROW, ROW)],
                                    stage_v.at[pl.ds(0, ROW)])
                    pltpu.sync_copy(stage_v.at[pl.ds(0, ROW)],
                                    out_hbm.at[pl.ds(b_s * ROW, ROW)])
                    np0, np1 = p0, p1

                return (mask & jnp.logical_not(sel), np0, np1)

            mask_f, p0f, p1f = lax.fori_loop(
                0, nm, match_body, (mask0, jnp.int32(0), jnp.int32(0)))

            @pl.when(p0f == 1)
            def _():
                pltpu.make_async_copy(stage_v.at[pl.ds(0, ROW)],
                                      out_hbm.at[pl.ds(0, ROW)], semA).wait()

            @pl.when(p1f == 1)
            def _():
                pltpu.make_async_copy(stage_v.at[pl.ds(SST, ROW)],
                                      out_hbm.at[pl.ds(0, ROW)], semB).wait()

            return pend2

        lax.fori_loop(0, nchunks, chunk_body, 0)

    _fire_slabs(embT, b0T, b1T, b2T, b3T, lo, tile0, ball0, semt0)

    def col_body(m, carry):
        is_tail_col = m >= nfull
        par = (m - lo) & 1
        nxt = m + 1
        do_pre = (nxt < hi) & (nxt < nfull)

        @pl.when(jnp.logical_not(is_tail_col) & (par == 0))
        def _():
            _wait_slabs(embT, b0T, b1T, b2T, b3T, tile0, ball0, semt0)

        @pl.when(jnp.logical_not(is_tail_col) & (par == 1))
        def _():
            _wait_slabs(embT, b0T, b1T, b2T, b3T, tile1, ball1, semt1)

        @pl.when(do_pre & (par == 0))
        def _():
            _fire_slabs(embT, b0T, b1T, b2T, b3T, nxt, tile1, ball1, semt1)

        @pl.when(do_pre & (par == 1))
        def _():
            _fire_slabs(embT, b0T, b1T, b2T, b3T, nxt, tile0, ball0, semt0)

        @pl.when(is_tail_col)
        def _():
            chunk_scan(m, True, tile0, ball0)

        @pl.when(jnp.logical_not(is_tail_col) & (par == 0))
        def _():
            chunk_scan(m, False, tile0, ball0)

        @pl.when(jnp.logical_not(is_tail_col) & (par == 1))
        def _():
            chunk_scan(m, False, tile1, ball1)

        return carry

    lax.fori_loop(lo, hi, col_body, 0)


def _phase1_body(user_hbm, item_hbm, uembT, iembT,
                 urbT, uvbT, uibT, ucfT, irbT, ivbT, iibT, icfT,
                 utail_hbm, itail_hbm, uout_hbm, iout_hbm,
                 uidx_v, iidx_v, cand_b, cand_u,
                 tile0, ball0, tile1, ball1, stage_v,
                 semt0, semt1, semA, semB):
    wid = lax.axis_index("s") * NC + lax.axis_index("c")

    pltpu.sync_copy(user_hbm, uidx_v)
    pltpu.sync_copy(item_hbm, iidx_v)

    _gather_side(uidx_v, uembT, urbT, uvbT, uibT, ucfT, utail_hbm, uout_hbm,
                 cand_b, cand_u, tile0, ball0, tile1, ball1,
                 stage_v, semt0, semt1, semA, semB,
                 wid, NCOL_U, NFULL_U, TAILB_U)
    _gather_side(iidx_v, iembT, irbT, ivbT, iibT, icfT, itail_hbm, iout_hbm,
                 cand_b, cand_u, tile0, ball0, tile1, ball1,
                 stage_v, semt0, semt1, semA, semB,
                 wid, NCOL_I, NFULL_I, TAILB_I)


def _phase2_body(uout_hbm, iout_hbm, g_hbm, out_hbm,
                 urows_v, irows_v, g_v, out_v, sem):
    wid = lax.axis_index("s") * NC + lax.axis_index("c")
    base = wid * BPW

    cps = [
        pltpu.make_async_copy(uout_hbm.at[pl.ds(base * ROW, BPW * ROW)],
                              urows_v, sem),
        pltpu.make_async_copy(iout_hbm.at[pl.ds(base * ROW, BPW * ROW)],
                              irows_v, sem),
        pltpu.make_async_copy(g_hbm, g_v, sem),
    ]
    for cp in cps:
        cp.start()
    for cp in cps:
        cp.wait()

    lanes = lax.iota(jnp.int32, L)

    def group_body(g, carry):
        jbase = (lanes + g * L) * ROW

        def tag_body(t, racc):
            tcol = jnp.full((L,), t, dtype=jnp.int32)
            dot = jnp.zeros((L,), jnp.float32)
            for e in range(EMB):
                col = jbase + tcol * EMB + e
                uv = plsc.load_gather(urows_v, [col])
                iv = plsc.load_gather(irows_v, [col])
                dot = dot + uv * iv

            def gscalar(k):
                krow = jnp.full((L,), k, dtype=jnp.int32)
                return plsc.load_gather(g_v, [krow * TAG_NUM + tcol])

            score = jnp.zeros((L,), jnp.float32)
            for k in range(3):
                bcol = jbase + F + TAG_NUM * k + tcol
                x = (dot + plsc.load_gather(urows_v, [bcol])
                     + plsc.load_gather(irows_v, [bcol]) + gscalar(k))
                score = score + 1.0 / (1.0 + jnp.exp(-x))
            score = score * (1.0 / 3.0)

            ccol = jbase + F + TAG_NUM * 3 + tcol
            coeff = (plsc.load_gather(urows_v, [ccol])
                     + plsc.load_gather(irows_v, [ccol]) + gscalar(3))
            return racc + score * coeff

        racc = lax.fori_loop(0, TAG_NUM, tag_body, jnp.zeros((L,), jnp.float32))
        out_v[pl.ds(g * L, L)] = racc
        return carry

    lax.fori_loop(0, BPW // L, group_body, 0)
    pltpu.sync_copy(out_v, out_hbm.at[pl.ds(base, BPW)])


def _make_phase1():
    mesh = plsc.VectorSubcoreMesh(core_axis_name="c", subcore_axis_name="s")
    return pl.kernel(
        _phase1_body,
        out_type=(jax.ShapeDtypeStruct((BATCH * ROW,), jnp.float32),
                  jax.ShapeDtypeStruct((BATCH * ROW,), jnp.float32)),
        mesh=mesh,
        scratch_types=[
            pltpu.VMEM((BATCH,), jnp.int32),
            pltpu.VMEM((BATCH,), jnp.int32),
            pltpu.VMEM((BATCH,), jnp.int32),
            pltpu.VMEM((BATCH,), jnp.int32),
            pltpu.VMEM((F, 128), jnp.float32),
            pltpu.VMEM((96, 128), jnp.float32),
            pltpu.VMEM((F, 128), jnp.float32),
            pltpu.VMEM((96, 128), jnp.float32),
            pltpu.VMEM((2 * SST,), jnp.float32),
            pltpu.SemaphoreType.DMA,
            pltpu.SemaphoreType.DMA,
            pltpu.SemaphoreType.DMA,
            pltpu.SemaphoreType.DMA,
        ],
        compiler_params=pltpu.CompilerParams(
            use_tc_tiling_on_sc=True, needs_layout_passes=False),
    )


def _make_phase2():
    mesh = plsc.VectorSubcoreMesh(core_axis_name="c", subcore_axis_name="s")
    return pl.kernel(
        _phase2_body,
        out_type=jax.ShapeDtypeStruct((BATCH,), jnp.float32),
        mesh=mesh,
        scratch_types=[
            pltpu.VMEM((BPW * ROW,), jnp.float32),
            pltpu.VMEM((BPW * ROW,), jnp.float32),
            pltpu.VMEM((4 * TAG_NUM,), jnp.float32),
            pltpu.VMEM((BPW,), jnp.float32),
            pltpu.SemaphoreType.DMA,
        ],
        compiler_params=pltpu.CompilerParams(
            use_tc_tiling_on_sc=False, needs_layout_passes=False),
    )


@jax.jit
def _sulm(user, item, uemb3, iemb3, urb, irb, uvb, ivb, uib, iib, ucf, icf, g4):
    uembT = jnp.transpose(uemb3, (1, 2, 0)).reshape(F, USER_NUM)
    iembT = jnp.transpose(iemb3, (1, 2, 0)).reshape(F, ITEM_NUM)

    def tails(emb3, b0, b1, b2, b3, tb, n):
        return jnp.concatenate(
            [emb3[tb:].reshape(n - tb, F), b0[tb:], b1[tb:], b2[tb:], b3[tb:]],
            axis=1).reshape(-1)

    utail = tails(uemb3, urb, uvb, uib, ucf, TAILB_U, USER_NUM)
    itail = tails(iemb3, irb, ivb, iib, icf, TAILB_I, ITEM_NUM)

    uout, iout = _make_phase1()(
        user, item, uembT, iembT,
        urb.T, uvb.T, uib.T, ucf.T, irb.T, ivb.T, iib.T, icf.T,
        utail, itail)
    return _make_phase2()(uout, iout, g4.reshape(-1))


def kernel(user, item,
           user_tag_embeddings, item_tag_embeddings,
           user_reason_bias, item_reason_bias, global_reason_bias,
           user_video_bias, item_video_bias, global_video_bias,
           user_interest_bias, item_interest_bias, global_interest_bias,
           user_coeff, item_coeff, global_coeff):
    g4 = jnp.concatenate(
        [global_reason_bias, global_video_bias, global_interest_bias,
         global_coeff], axis=0)
    return _sulm(user, item, user_tag_embeddings, item_tag_embeddings,
                 user_reason_bias, item_reason_bias,
                 user_video_bias, item_video_bias,
                 user_interest_bias, item_interest_bias,
                 user_coeff, item_coeff, g4)
